# Initial kernel scaffold; baseline (speedup 1.0000x reference)
import functools

import jax
import jax.numpy as jnp
from jax import lax
from jax.experimental import pallas as pl
from jax.experimental.pallas import tpu as pltpu
from jax.experimental.pallas import tpu_sc as plsc

N_USERS = 25000
N_ITEMS = 25000
HALF = 25000
N_NODES = N_USERS + N_ITEMS
D = 64
NUM_LAYERS = 3
E = 800000

NTILES = 16
ROWS_PT = 1568
PAD_HALF = NTILES * ROWS_PT
PADN = 2 * PAD_HALF
PAD_OFF = PAD_HALF - HALF

CH = 128
PER_TILE_E = 50048
NCH = PER_TILE_E // CH
EPAD = NTILES * PER_TILE_E

BLK = 224
NBLK = ROWS_PT // BLK

_mesh = plsc.VectorSubcoreMesh(core_axis_name="c", subcore_axis_name="s")


def _zeros16():
    return jnp.zeros((16,), jnp.float32)


@functools.partial(
    pl.kernel,
    mesh=_mesh,
    out_type=jax.ShapeDtypeStruct((PADN,), jnp.float32),
    scratch_types=[
        pltpu.VMEM((CH,), jnp.int32),
        pltpu.VMEM((CH,), jnp.int32),
        pltpu.VMEM((CH, 16), jnp.float32),
        pltpu.VMEM((ROWS_PT, 16), jnp.float32),
        pltpu.VMEM((ROWS_PT,), jnp.float32),
        pltpu.VMEM_SHARED((PAD_HALF, 16), jnp.float32),
    ],
)
def _deg_kernel(col_hbm, dinv_hbm, col_v, loc_v, ones_v, dstage, dout, accd):
    sc = lax.axis_index("c")
    t = lax.axis_index("s")
    base_dst = sc * HALF

    def init_body(i, carry):
        ones_v[i, pl.ds(0, 16)] = jnp.ones((16,), jnp.float32)
        dstage_row = i
        return carry

    lax.fori_loop(0, CH, init_body, 0)

    def zstage_body(i, carry):
        dstage[i, pl.ds(0, 16)] = _zeros16()
        return carry

    lax.fori_loop(0, ROWS_PT, zstage_body, 0)
    pltpu.sync_copy(dstage, accd.at[pl.ds(t * ROWS_PT, ROWS_PT)])
    plsc.subcore_barrier()

    ebase = t * PER_TILE_E
    half16 = jnp.full((16,), HALF, jnp.int32)
    zero16 = jnp.zeros((16,), jnp.int32)

    def chunk_body(cidx, carry):
        off = ebase + cidx * CH
        pltpu.sync_copy(col_hbm.at[pl.ds(off, CH)], col_v)
        for j in range(CH // 16):
            sl = pl.ds(j * 16, 16)
            c = col_v[sl]
            local = c - base_dst
            ok = (local >= zero16) & (local < half16)
            loc_v[sl] = jnp.where(ok, local, half16)
        pltpu.sync_copy(ones_v, accd.at[loc_v], add=True)
        return carry

    lax.fori_loop(0, NCH, chunk_body, 0)
    plsc.subcore_barrier()

    pltpu.sync_copy(accd.at[pl.ds(t * ROWS_PT, ROWS_PT)], dstage)
    lane = lax.iota(jnp.int32, 16)
    zcol = jnp.zeros((16,), jnp.int32)
    onef = jnp.ones((16,), jnp.float32)
    zerof = _zeros16()

    def inv_body(i, carry):
        d = plsc.load_gather(dstage, [i * 16 + lane, zcol])
        dout[pl.ds(i * 16, 16)] = jnp.where(d > zerof, onef / d, zerof)
        return carry

    lax.fori_loop(0, ROWS_PT // 16, inv_body, 0)
    pltpu.sync_copy(dout, dinv_hbm.at[pl.ds(sc * PAD_HALF + t * ROWS_PT, ROWS_PT)])


@functools.partial(
    pl.kernel,
    mesh=_mesh,
    out_type=jax.ShapeDtypeStruct((PADN, D), jnp.float32),
    scratch_types=[
        pltpu.VMEM((CH,), jnp.int32),
        pltpu.VMEM((CH,), jnp.int32),
        pltpu.VMEM((CH,), jnp.int32),
        pltpu.VMEM((CH, D), jnp.float32),
        pltpu.VMEM((BLK, D), jnp.float32),
        pltpu.VMEM((BLK,), jnp.float32),
        pltpu.SemaphoreType.DMA,
        pltpu.VMEM_SHARED((PAD_HALF, D), jnp.float32),
    ],
)
def _layer_kernel(row_hbm, col_hbm, dinv_hbm, x_hbm, out_hbm,
                  rowm_v, col_v, loc_v, rows_v, buf, dbuf, sem, acc):
    sc = lax.axis_index("c")
    t = lax.axis_index("s")
    base_dst = sc * HALF

    def zbuf_body(i, carry):
        for j in range(D // 16):
            buf[i, pl.ds(j * 16, 16)] = _zeros16()
        return carry

    lax.fori_loop(0, BLK, zbuf_body, 0)

    def zacc_body(b, carry):
        pltpu.sync_copy(buf, acc.at[pl.ds(t * ROWS_PT + b * BLK, BLK)])
        return carry

    lax.fori_loop(0, NBLK, zacc_body, 0)
    plsc.subcore_barrier()

    ebase = t * PER_TILE_E
    half16 = jnp.full((16,), HALF, jnp.int32)
    zero16 = jnp.zeros((16,), jnp.int32)
    shift16 = jnp.full((16,), PAD_OFF, jnp.int32)

    def chunk_body(cidx, carry):
        off = ebase + cidx * CH
        pltpu.sync_copy(row_hbm.at[pl.ds(off, CH)], rowm_v)
        pltpu.sync_copy(col_hbm.at[pl.ds(off, CH)], col_v)
        for j in range(CH // 16):
            sl = pl.ds(j * 16, 16)
            r = rowm_v[sl]
            rowm_v[sl] = jnp.where(r >= half16, r + shift16, r)
            c = col_v[sl]
            local = c - base_dst
            ok = (local >= zero16) & (local < half16)
            loc_v[sl] = jnp.where(ok, local, half16)
        pltpu.async_copy(x_hbm.at[rowm_v], rows_v, sem).wait()
        pltpu.sync_copy(rows_v, acc.at[loc_v], add=True)
        return carry

    lax.fori_loop(0, NCH, chunk_body, 0)
    plsc.subcore_barrier()

    def scale_blk(b, carry):
        roff = t * ROWS_PT + b * BLK
        pltpu.sync_copy(acc.at[pl.ds(roff, BLK)], buf)
        pltpu.sync_copy(dinv_hbm.at[pl.ds(sc * PAD_HALF + roff, BLK)], dbuf)

        def scale_row(r, c2):
            dv = plsc.load_gather(dbuf, [jnp.full((16,), 0, jnp.int32) + r])
            for j in range(D // 16):
                sl = pl.ds(j * 16, 16)
                buf[r, sl] = buf[r, sl] * dv
            return c2

        lax.fori_loop(0, BLK, scale_row, 0)
        pltpu.sync_copy(buf, out_hbm.at[pl.ds(sc * PAD_HALF + roff, BLK)])
        return carry

    lax.fori_loop(0, NBLK, scale_blk, 0)


def _mean_body(a, b, c, d, o):
    o[...] = (a[...] + b[...] + c[...] + d[...]) * 0.25


def _mean4(x0, x1, x2, x3):
    n = PADN * D // 128
    blk = (n // 16, 128)
    spec = pl.BlockSpec(blk, lambda i: (i, 0))
    f = pl.pallas_call(
        _mean_body,
        grid=(16,),
        in_specs=[spec] * 4,
        out_specs=spec,
        out_shape=jax.ShapeDtypeStruct((n, 128), jnp.float32),
    )
    r = lambda x: x.reshape(n, 128)
    return f(r(x0), r(x1), r(x2), r(x3)).reshape(PADN, D)


@jax.jit
def kernel(edge_index, user_emb, item_emb):
    row = edge_index[0]
    col = edge_index[1]
    rowp = jnp.pad(row, (0, EPAD - E))
    colp = jnp.pad(col, (0, EPAD - E), constant_values=N_NODES)

    x0 = jnp.zeros((PADN, D), jnp.float32)
    x0 = x0.at[0:HALF].set(user_emb)
    x0 = x0.at[PAD_HALF:PAD_HALF + HALF].set(item_emb)

    dinv = _deg_kernel(colp)
    x1 = _layer_kernel(rowp, colp, dinv, x0)
    x2 = _layer_kernel(rowp, colp, dinv, x1)
    x3 = _layer_kernel(rowp, colp, dinv, x2)

    m = _mean4(x0, x1, x2, x3)
    return m[0:HALF], m[PAD_HALF:PAD_HALF + HALF]


# speedup vs baseline: 4.6982x; 4.6982x over previous
import functools

import jax
import jax.numpy as jnp
from jax import lax
from jax.experimental import pallas as pl
from jax.experimental.pallas import tpu as pltpu
from jax.experimental.pallas import tpu_sc as plsc

N_USERS = 25000
N_ITEMS = 25000
HALF = 25000
N_NODES = N_USERS + N_ITEMS
D = 64
NUM_LAYERS = 3
E = 800000

NTILES = 16
ROWS_PT = 1568
PAD_HALF = NTILES * ROWS_PT
PADN = 2 * PAD_HALF
PAD_OFF = PAD_HALF - HALF

CH = 128
CPS = 8
SLAB = CH * CPS
NSLAB = 49
PER_TILE_E = NSLAB * SLAB
EPAD = NTILES * PER_TILE_E
EROWS_PT = PER_TILE_E // CH

NBUF = 3

BLK = 112
NBLK = ROWS_PT // BLK

_mesh = plsc.VectorSubcoreMesh(core_axis_name="c", subcore_axis_name="s")
_sc_params = pltpu.CompilerParams(use_tc_tiling_on_sc=False)


def _zeros16():
    return jnp.zeros((16,), jnp.float32)


@functools.partial(
    pl.kernel,
    mesh=_mesh,
    out_type=jax.ShapeDtypeStruct((PADN, 16), jnp.float32),
    scratch_types=[
        pltpu.VMEM((CPS, CH), jnp.int32),
        pltpu.VMEM((CH, 16), jnp.float32),
        pltpu.VMEM((ROWS_PT, 16), jnp.float32),
        pltpu.SemaphoreType.DMA,
        pltpu.VMEM_SHARED((PAD_HALF, 16), jnp.float32),
    ],
    compiler_params=_sc_params,
)
def _deg_kernel(col_hbm, dinv_hbm, loc2, ones_v, dstage, ssem, accd):
    sc = lax.axis_index("c")
    t = lax.axis_index("s")
    base_dst = sc * HALF

    def init_body(i, carry):
        ones_v[i, pl.ds(0, 16)] = jnp.ones((16,), jnp.float32)
        return carry

    lax.fori_loop(0, CH, init_body, 0)

    def zstage_body(i, carry):
        dstage[i, pl.ds(0, 16)] = _zeros16()
        return carry

    lax.fori_loop(0, ROWS_PT, zstage_body, 0)
    pltpu.sync_copy(dstage, accd.at[pl.ds(t * ROWS_PT, ROWS_PT)])
    plsc.subcore_barrier()

    erow0 = t * EROWS_PT
    half16 = jnp.full((16,), HALF, jnp.int32)
    zero16 = jnp.zeros((16,), jnp.int32)
    nsub = CH // 16

    def slab_body(s, carry):
        pltpu.sync_copy(col_hbm.at[pl.ds(erow0 + s * CPS, CPS)], loc2)
        for j in range(SLAB // 16):
            dst = (j // nsub, pl.ds((j % nsub) * 16, 16))
            local = loc2[dst] - base_dst
            ok = (local >= zero16) & (local < half16)
            loc2[dst] = jnp.where(ok, local, half16)
        hs = [pltpu.async_copy(ones_v, accd.at[loc2.at[k]], ssem, add=True)
              for k in range(CPS)]
        for h in hs:
            h.wait()
        return carry

    lax.fori_loop(0, NSLAB, slab_body, 0)
    plsc.subcore_barrier()

    pltpu.sync_copy(accd.at[pl.ds(t * ROWS_PT, ROWS_PT)], dstage)
    onef = jnp.ones((16,), jnp.float32)
    zerof = _zeros16()

    def inv_body(i, carry):
        d = dstage[i, pl.ds(0, 16)]
        dstage[i, pl.ds(0, 16)] = jnp.where(d > zerof, onef / d, zerof)
        return carry

    lax.fori_loop(0, ROWS_PT, inv_body, 0)
    pltpu.sync_copy(dstage, dinv_hbm.at[pl.ds(sc * PAD_HALF + t * ROWS_PT, ROWS_PT)])


@functools.partial(
    pl.kernel,
    mesh=_mesh,
    out_type=jax.ShapeDtypeStruct((PADN, D), jnp.float32),
    scratch_types=[
        pltpu.VMEM((CPS, CH), jnp.int32),
        pltpu.VMEM((CPS, CH), jnp.int32),
        [pltpu.VMEM((CH, D), jnp.float32) for _ in range(NBUF)],
        pltpu.VMEM((BLK, 16), jnp.float32),
        pltpu.SemaphoreType.DMA,
        pltpu.SemaphoreType.DMA,
        pltpu.VMEM_SHARED((PAD_HALF, D), jnp.float32),
    ],
    compiler_params=_sc_params,
)
def _layer_kernel(row_hbm, col_hbm, dinv_hbm, x_hbm, out_hbm,
                  rowm2, loc2, rbufs, dbuf, gsem, ssem, acc):
    sc = lax.axis_index("c")
    t = lax.axis_index("s")
    base_dst = sc * HALF
    buf = rbufs[0]

    def zbuf_body(i, carry):
        for j in range(D // 16):
            buf[i, pl.ds(j * 16, 16)] = _zeros16()
        return carry

    lax.fori_loop(0, BLK, zbuf_body, 0)

    def zacc_body(b, carry):
        pltpu.sync_copy(buf.at[pl.ds(0, BLK)],
                        acc.at[pl.ds(t * ROWS_PT + b * BLK, BLK)])
        return carry

    lax.fori_loop(0, NBLK, zacc_body, 0)
    plsc.subcore_barrier()

    erow0 = t * EROWS_PT
    half16 = jnp.full((16,), HALF, jnp.int32)
    zero16 = jnp.zeros((16,), jnp.int32)
    shift16 = jnp.full((16,), PAD_OFF, jnp.int32)
    nsub = CH // 16

    def slab_body(s, carry):
        pltpu.sync_copy(row_hbm.at[pl.ds(erow0 + s * CPS, CPS)], rowm2)
        pltpu.sync_copy(col_hbm.at[pl.ds(erow0 + s * CPS, CPS)], loc2)
        for j in range(SLAB // 16):
            dst = (j // nsub, pl.ds((j % nsub) * 16, 16))
            r = rowm2[dst]
            rowm2[dst] = jnp.where(r >= half16, r + shift16, r)
            local = loc2[dst] - base_dst
            ok = (local >= zero16) & (local < half16)
            loc2[dst] = jnp.where(ok, local, half16)
        ghs = [pltpu.async_copy(x_hbm.at[rowm2.at[k]], rbufs[k], gsem)
               for k in range(NBUF)]
        shs = []
        for k in range(CPS):
            ghs[k].wait()
            shs.append(pltpu.async_copy(rbufs[k % NBUF], acc.at[loc2.at[k]],
                                        ssem, add=True))
            if k + NBUF < CPS:
                shs[k].wait()
                ghs.append(pltpu.async_copy(x_hbm.at[rowm2.at[k + NBUF]],
                                            rbufs[k % NBUF], gsem))
        for h in shs[CPS - NBUF:]:
            h.wait()
        return carry

    lax.fori_loop(0, NSLAB, slab_body, 0)
    plsc.subcore_barrier()

    def scale_blk(b, carry):
        roff = t * ROWS_PT + b * BLK
        pltpu.sync_copy(acc.at[pl.ds(roff, BLK)], buf.at[pl.ds(0, BLK)])
        pltpu.sync_copy(dinv_hbm.at[pl.ds(sc * PAD_HALF + roff, BLK)], dbuf)

        def scale_row(r, c2):
            dv = dbuf[r, pl.ds(0, 16)]
            for j in range(D // 16):
                sl = pl.ds(j * 16, 16)
                buf[r, sl] = buf[r, sl] * dv
            return c2

        lax.fori_loop(0, BLK, scale_row, 0)
        pltpu.sync_copy(buf.at[pl.ds(0, BLK)],
                        out_hbm.at[pl.ds(sc * PAD_HALF + roff, BLK)])
        return carry

    lax.fori_loop(0, NBLK, scale_blk, 0)


def _mean_body(a, b, c, d, o):
    o[...] = (a[...] + b[...] + c[...] + d[...]) * 0.25


def _mean4(x0, x1, x2, x3):
    n = PADN * D // 128
    blk = (n // 16, 128)
    spec = pl.BlockSpec(blk, lambda i: (i, 0))
    f = pl.pallas_call(
        _mean_body,
        grid=(16,),
        in_specs=[spec] * 4,
        out_specs=spec,
        out_shape=jax.ShapeDtypeStruct((n, 128), jnp.float32),
    )
    r = lambda x: x.reshape(n, 128)
    return f(r(x0), r(x1), r(x2), r(x3)).reshape(PADN, D)


@jax.jit
def kernel(edge_index, user_emb, item_emb):
    row = edge_index[0]
    col = edge_index[1]
    rowp = jnp.pad(row, (0, EPAD - E)).reshape(EPAD // CH, CH)
    colp = jnp.pad(col, (0, EPAD - E),
                   constant_values=N_NODES).reshape(EPAD // CH, CH)

    x0 = jnp.zeros((PADN, D), jnp.float32)
    x0 = x0.at[0:HALF].set(user_emb)
    x0 = x0.at[PAD_HALF:PAD_HALF + HALF].set(item_emb)

    dinv = _deg_kernel(colp)
    x1 = _layer_kernel(rowp, colp, dinv, x0)
    x2 = _layer_kernel(rowp, colp, dinv, x1)
    x3 = _layer_kernel(rowp, colp, dinv, x2)

    m = _mean4(x0, x1, x2, x3)
    return m[0:HALF], m[PAD_HALF:PAD_HALF + HALF]
